# baseline (speedup 1.0000x reference)
import functools

import jax
import jax.numpy as jnp
from jax import lax
from jax.experimental import pallas as pl
from jax.experimental.pallas import tpu as pltpu
from jax.experimental.pallas import tpu_sc as plsc

B = 16384
NCHUNK = 2
BC = B // NCHUNK
EMB = 128
LAT = 64
CHUNK = 128
STAGE = 256
NBUF = 3

_NC, _NS = 2, 16
_NW = _NC * _NS
_BPW = BC // _NW
_NSTAGE = 3 * (_BPW // STAGE)


@functools.cache
def _make_gather3(chunk_base):
    mesh = plsc.VectorSubcoreMesh(
        core_axis_name="c", subcore_axis_name="s", num_cores=_NC
    )

    @functools.partial(
        pl.kernel,
        mesh=mesh,
        out_type=(
            jax.ShapeDtypeStruct((BC, EMB), jnp.float32),
            jax.ShapeDtypeStruct((BC, EMB), jnp.float32),
            jax.ShapeDtypeStruct((BC, EMB), jnp.float32),
        ),
        scratch_types=[
            pltpu.VMEM((_BPW,), jnp.int32),
            pltpu.VMEM((_BPW,), jnp.int32),
            pltpu.VMEM((_BPW,), jnp.int32),
            pltpu.VMEM((STAGE, EMB), jnp.float32),
            pltpu.VMEM((STAGE, EMB), jnp.float32),
            pltpu.VMEM((STAGE, EMB), jnp.float32),
            pltpu.SemaphoreType.DMA,
            pltpu.SemaphoreType.DMA,
            pltpu.SemaphoreType.DMA,
        ],
    )
    def gather3(
        g_hbm, b_hbm, p_hbm, gt_hbm, bt_hbm, pt_hbm,
        og_hbm, ob_hbm, op_hbm, ig_v, ib_v, ip_v, r0_v, r1_v, r2_v,
        isem, gsem, wsem,
    ):
        wid = lax.axis_index("s") * _NC + lax.axis_index("c")
        base = wid * _BPW
        tabs = (gt_hbm, bt_hbm, pt_hbm)
        outs = (og_hbm, ob_hbm, op_hbm)
        idxs = (ig_v, ib_v, ip_v)
        bufs = (r0_v, r1_v, r2_v)
        icopies = [
            pltpu.async_copy(idx.at[pl.ds(chunk_base + base, _BPW)], idxs[t], isem)
            for t, idx in enumerate((g_hbm, b_hbm, p_hbm))
        ]
        for c in icopies:
            c.wait()

        spt = _BPW // STAGE

        def fire_gather(s):
            t, h = s // spt, s % spt
            return [
                pltpu.async_copy(
                    tabs[t].at[idxs[t].at[pl.ds(h * STAGE + j * CHUNK, CHUNK)]],
                    bufs[s % NBUF].at[pl.ds(j * CHUNK, CHUNK)],
                    gsem,
                )
                for j in range(STAGE // CHUNK)
            ]

        def fire_write(s):
            t, h = s // spt, s % spt
            return pltpu.async_copy(
                bufs[s % NBUF],
                outs[t].at[pl.ds(base + h * STAGE, STAGE)],
                wsem,
            )

        gathers = {0: fire_gather(0)}
        writes = {}
        for s in range(_NSTAGE):
            if s + 1 < _NSTAGE:
                if s + 1 >= NBUF:
                    writes[s + 1 - NBUF].wait()
                gathers[s + 1] = fire_gather(s + 1)
            for c in gathers[s]:
                c.wait()
            writes[s] = fire_write(s)
        for s in range(max(0, _NSTAGE - NBUF), _NSTAGE):
            writes[s].wait()

    return gather3


BLK = 4096


def _proj_body(xg_ref, xb_ref, xp_ref, wp_ref, bp_ref, wf_ref, bf_ref, o_ref):
    h = jnp.dot(xg_ref[...], wp_ref[0:EMB, :], preferred_element_type=jnp.float32)
    h = h + jnp.dot(xb_ref[...], wp_ref[EMB : 2 * EMB, :], preferred_element_type=jnp.float32)
    h = h + jnp.dot(xp_ref[...], wp_ref[2 * EMB : 3 * EMB, :], preferred_element_type=jnp.float32)
    h = h + bp_ref[...]
    ot = lax.dot_general(
        wf_ref[...], h, (((0,), (1,)), ((), ())),
        preferred_element_type=jnp.float32,
    )
    o_ref[...] = ot + bf_ref[...]


def _proj_body_acc(xg_ref, xb_ref, xp_ref, wp_ref, bp_ref, wf_ref, bf_ref, acc_ref, o_ref):
    del acc_ref
    _proj_body(xg_ref, xb_ref, xp_ref, wp_ref, bp_ref, wf_ref, bf_ref, o_ref)


def _proj_chunk(c, xg, xb, xp, W_proj, b_proj, W_fc, b_fc, acc):
    nblk = BC // BLK
    specs = [
        pl.BlockSpec((BLK, EMB), lambda i: (i, 0)),
        pl.BlockSpec((BLK, EMB), lambda i: (i, 0)),
        pl.BlockSpec((BLK, EMB), lambda i: (i, 0)),
        pl.BlockSpec((3 * EMB, EMB), lambda i: (0, 0)),
        pl.BlockSpec((1, EMB), lambda i: (0, 0)),
        pl.BlockSpec((EMB, LAT), lambda i: (0, 0)),
        pl.BlockSpec((LAT, 1), lambda i: (0, 0)),
    ]
    args = [xg, xb, xp, W_proj, b_proj.reshape(1, EMB), W_fc, b_fc.reshape(LAT, 1)]
    body = _proj_body
    aliases = {}
    if acc is not None:
        specs.append(pl.BlockSpec(memory_space=pl.ANY))
        args.append(acc)
        body = _proj_body_acc
        aliases = {7: 0}
    return pl.pallas_call(
        body,
        grid=(nblk,),
        in_specs=specs,
        out_specs=pl.BlockSpec((LAT, BLK), lambda i, c=c: (0, c * (BC // BLK) + i)),
        out_shape=jax.ShapeDtypeStruct((LAT, B), jnp.float32),
        input_output_aliases=aliases,
    )(*args)


def kernel(gains, bws, pms, gain_table, bw_table, pm_table, W_proj, b_proj, W_fc, b_fc):
    g = gains.astype(jnp.int32)
    bw = bws.astype(jnp.int32)
    pm = pms.astype(jnp.int32)
    chunks = [
        _make_gather3(c * BC)(g, bw, pm, gain_table, bw_table, pm_table)
        for c in range(NCHUNK)
    ]
    acc = None
    for c, (xg, xb, xp) in enumerate(chunks):
        acc = _proj_chunk(c, xg, xb, xp, W_proj, b_proj, W_fc, b_fc, acc)
    return acc.T


# speedup vs baseline: 1.0283x; 1.0283x over previous
import functools

import jax
import jax.numpy as jnp
from jax import lax
from jax.experimental import pallas as pl
from jax.experimental.pallas import tpu as pltpu
from jax.experimental.pallas import tpu_sc as plsc

B = 16384
CHUNK_ROWS = (16384,)
EMB = 128
LAT = 64
CHUNK = 128
STAGE = 256
NBUF = 3

_NC, _NS = 2, 16
_NW = _NC * _NS


@functools.cache
def _make_gather3(chunk_base, bc):
    bpw = bc // _NW
    nstage = 3 * (bpw // STAGE)
    spt = bpw // STAGE
    mesh = plsc.VectorSubcoreMesh(
        core_axis_name="c", subcore_axis_name="s", num_cores=_NC
    )

    @functools.partial(
        pl.kernel,
        mesh=mesh,
        out_type=(
            jax.ShapeDtypeStruct((bc, EMB), jnp.float32),
            jax.ShapeDtypeStruct((bc, EMB), jnp.float32),
            jax.ShapeDtypeStruct((bc, EMB), jnp.float32),
        ),
        scratch_types=[
            pltpu.VMEM((bpw,), jnp.int32),
            pltpu.VMEM((bpw,), jnp.int32),
            pltpu.VMEM((bpw,), jnp.int32),
            pltpu.VMEM((STAGE, EMB), jnp.float32),
            pltpu.VMEM((STAGE, EMB), jnp.float32),
            pltpu.VMEM((STAGE, EMB), jnp.float32),
            pltpu.SemaphoreType.DMA,
            pltpu.SemaphoreType.DMA,
            pltpu.SemaphoreType.DMA,
        ],
    )
    def gather3(
        g_hbm, b_hbm, p_hbm, gt_hbm, bt_hbm, pt_hbm,
        og_hbm, ob_hbm, op_hbm, ig_v, ib_v, ip_v, r0_v, r1_v, r2_v,
        isem, gsem, wsem,
    ):
        wid = lax.axis_index("s") * _NC + lax.axis_index("c")
        base = wid * bpw
        tabs = (gt_hbm, bt_hbm, pt_hbm)
        outs = (og_hbm, ob_hbm, op_hbm)
        idxs = (ig_v, ib_v, ip_v)
        bufs = (r0_v, r1_v, r2_v)
        icopies = [
            pltpu.async_copy(idx.at[pl.ds(chunk_base + base, bpw)], idxs[t], isem)
            for t, idx in enumerate((g_hbm, b_hbm, p_hbm))
        ]
        idx_ready = [False, False, False]

        def fire_gather(s):
            t, h = s // spt, s % spt
            if not idx_ready[t]:
                icopies[t].wait()
                idx_ready[t] = True
            return [
                pltpu.async_copy(
                    tabs[t].at[idxs[t].at[pl.ds(h * STAGE + j * CHUNK, CHUNK)]],
                    bufs[s % NBUF].at[pl.ds(j * CHUNK, CHUNK)],
                    gsem,
                )
                for j in range(STAGE // CHUNK)
            ]

        def fire_write(s):
            t, h = s // spt, s % spt
            return pltpu.async_copy(
                bufs[s % NBUF],
                outs[t].at[pl.ds(base + h * STAGE, STAGE)],
                wsem,
            )

        gathers = {0: fire_gather(0)}
        writes = {}
        for s in range(nstage):
            if s + 1 < nstage:
                if s + 1 >= NBUF:
                    writes[s + 1 - NBUF].wait()
                gathers[s + 1] = fire_gather(s + 1)
            for c in gathers[s]:
                c.wait()
            writes[s] = fire_write(s)
        for s in range(max(0, nstage - NBUF), nstage):
            writes[s].wait()

    return gather3


BLK = 8192


def _proj_body(xg_ref, xb_ref, xp_ref, wp_ref, bp_ref, wf_ref, bf_ref, o_ref):
    h = jnp.dot(xg_ref[...], wp_ref[0:EMB, :], preferred_element_type=jnp.float32)
    h = h + jnp.dot(xb_ref[...], wp_ref[EMB : 2 * EMB, :], preferred_element_type=jnp.float32)
    h = h + jnp.dot(xp_ref[...], wp_ref[2 * EMB : 3 * EMB, :], preferred_element_type=jnp.float32)
    h = h + bp_ref[...]
    ot = lax.dot_general(
        wf_ref[...], h, (((0,), (1,)), ((), ())),
        preferred_element_type=jnp.float32,
    )
    o_ref[...] = ot + bf_ref[...]


def _proj_body_acc(xg_ref, xb_ref, xp_ref, wp_ref, bp_ref, wf_ref, bf_ref, acc_ref, o_ref):
    del acc_ref
    _proj_body(xg_ref, xb_ref, xp_ref, wp_ref, bp_ref, wf_ref, bf_ref, o_ref)


def _proj_chunk(col_base, xg, xb, xp, W_proj, b_proj, W_fc, b_fc, acc):
    bc = xg.shape[0]
    blk = min(BLK, bc)
    nblk = bc // blk
    specs = [
        pl.BlockSpec((blk, EMB), lambda i: (i, 0)),
        pl.BlockSpec((blk, EMB), lambda i: (i, 0)),
        pl.BlockSpec((blk, EMB), lambda i: (i, 0)),
        pl.BlockSpec((3 * EMB, EMB), lambda i: (0, 0)),
        pl.BlockSpec((1, EMB), lambda i: (0, 0)),
        pl.BlockSpec((EMB, LAT), lambda i: (0, 0)),
        pl.BlockSpec((LAT, 1), lambda i: (0, 0)),
    ]
    args = [xg, xb, xp, W_proj, b_proj.reshape(1, EMB), W_fc, b_fc.reshape(LAT, 1)]
    body = _proj_body
    aliases = {}
    if acc is not None:
        specs.append(pl.BlockSpec(memory_space=pl.ANY))
        args.append(acc)
        body = _proj_body_acc
        aliases = {7: 0}
    blk_base = col_base // blk
    return pl.pallas_call(
        body,
        grid=(nblk,),
        in_specs=specs,
        out_specs=pl.BlockSpec((LAT, blk), lambda i, b=blk_base: (0, b + i)),
        out_shape=jax.ShapeDtypeStruct((LAT, B), jnp.float32),
        input_output_aliases=aliases,
    )(*args)


def kernel(gains, bws, pms, gain_table, bw_table, pm_table, W_proj, b_proj, W_fc, b_fc):
    g = gains.astype(jnp.int32)
    bw = bws.astype(jnp.int32)
    pm = pms.astype(jnp.int32)
    chunks = []
    off = 0
    for bc in CHUNK_ROWS:
        chunks.append((off, _make_gather3(off, bc)(g, bw, pm, gain_table, bw_table, pm_table)))
        off += bc
    acc = None
    for off, (xg, xb, xp) in chunks:
        acc = _proj_chunk(off, xg, xb, xp, W_proj, b_proj, W_fc, b_fc, acc)
    return acc.T


# speedup vs baseline: 1.0646x; 1.0353x over previous
import functools

import jax
import jax.numpy as jnp
from jax import lax
from jax.experimental import pallas as pl
from jax.experimental.pallas import tpu as pltpu
from jax.experimental.pallas import tpu_sc as plsc

B = 16384
CHUNK_ROWS = (12288, 4096)
EMB = 128
LAT = 64
CHUNK = 128
STAGE = 256
NBUF = 3

_NC, _NS = 2, 16
_NW = _NC * _NS


@functools.cache
def _make_gather3(chunk_base, bc):
    bpw = bc // _NW
    stage = min(STAGE, bpw)
    nstage = 3 * (bpw // stage)
    spt = bpw // stage
    mesh = plsc.VectorSubcoreMesh(
        core_axis_name="c", subcore_axis_name="s", num_cores=_NC
    )

    @functools.partial(
        pl.kernel,
        mesh=mesh,
        out_type=(
            jax.ShapeDtypeStruct((bc, EMB), jnp.float32),
            jax.ShapeDtypeStruct((bc, EMB), jnp.float32),
            jax.ShapeDtypeStruct((bc, EMB), jnp.float32),
        ),
        scratch_types=[
            pltpu.VMEM((bpw,), jnp.int32),
            pltpu.VMEM((bpw,), jnp.int32),
            pltpu.VMEM((bpw,), jnp.int32),
            pltpu.VMEM((stage, EMB), jnp.float32),
            pltpu.VMEM((stage, EMB), jnp.float32),
            pltpu.VMEM((stage, EMB), jnp.float32),
            pltpu.SemaphoreType.DMA,
            pltpu.SemaphoreType.DMA,
            pltpu.SemaphoreType.DMA,
        ],
    )
    def gather3(
        g_hbm, b_hbm, p_hbm, gt_hbm, bt_hbm, pt_hbm,
        og_hbm, ob_hbm, op_hbm, ig_v, ib_v, ip_v, r0_v, r1_v, r2_v,
        isem, gsem, wsem,
    ):
        wid = lax.axis_index("s") * _NC + lax.axis_index("c")
        base = wid * bpw
        tabs = (gt_hbm, bt_hbm, pt_hbm)
        outs = (og_hbm, ob_hbm, op_hbm)
        idxs = (ig_v, ib_v, ip_v)
        bufs = (r0_v, r1_v, r2_v)
        icopies = [
            pltpu.async_copy(idx.at[pl.ds(chunk_base + base, bpw)], idxs[t], isem)
            for t, idx in enumerate((g_hbm, b_hbm, p_hbm))
        ]
        idx_ready = [False, False, False]

        def fire_gather(s):
            t, h = s // spt, s % spt
            if not idx_ready[t]:
                icopies[t].wait()
                idx_ready[t] = True
            return [
                pltpu.async_copy(
                    tabs[t].at[idxs[t].at[pl.ds(h * stage + j * CHUNK, CHUNK)]],
                    bufs[s % NBUF].at[pl.ds(j * CHUNK, CHUNK)],
                    gsem,
                )
                for j in range(stage // CHUNK)
            ]

        def fire_write(s):
            t, h = s // spt, s % spt
            return pltpu.async_copy(
                bufs[s % NBUF],
                outs[t].at[pl.ds(base + h * stage, stage)],
                wsem,
            )

        gathers = {0: fire_gather(0)}
        writes = {}
        for s in range(nstage):
            if s + 1 < nstage:
                if s + 1 >= NBUF:
                    writes[s + 1 - NBUF].wait()
                gathers[s + 1] = fire_gather(s + 1)
            for c in gathers[s]:
                c.wait()
            writes[s] = fire_write(s)
        for s in range(max(0, nstage - NBUF), nstage):
            writes[s].wait()

    return gather3


BLK = 4096


def _proj_body(xg_ref, xb_ref, xp_ref, wp_ref, bp_ref, wf_ref, bf_ref, o_ref):
    h = jnp.dot(xg_ref[...], wp_ref[0:EMB, :], preferred_element_type=jnp.float32)
    h = h + jnp.dot(xb_ref[...], wp_ref[EMB : 2 * EMB, :], preferred_element_type=jnp.float32)
    h = h + jnp.dot(xp_ref[...], wp_ref[2 * EMB : 3 * EMB, :], preferred_element_type=jnp.float32)
    h = h + bp_ref[...]
    ot = lax.dot_general(
        wf_ref[...], h, (((0,), (1,)), ((), ())),
        preferred_element_type=jnp.float32,
    )
    o_ref[...] = ot + bf_ref[...]


def _proj_body_acc(xg_ref, xb_ref, xp_ref, wp_ref, bp_ref, wf_ref, bf_ref, acc_ref, o_ref):
    del acc_ref
    _proj_body(xg_ref, xb_ref, xp_ref, wp_ref, bp_ref, wf_ref, bf_ref, o_ref)


def _proj_chunk(col_base, xg, xb, xp, W_proj, b_proj, W_fc, b_fc, acc):
    bc = xg.shape[0]
    blk = min(BLK, bc)
    nblk = bc // blk
    specs = [
        pl.BlockSpec((blk, EMB), lambda i: (i, 0)),
        pl.BlockSpec((blk, EMB), lambda i: (i, 0)),
        pl.BlockSpec((blk, EMB), lambda i: (i, 0)),
        pl.BlockSpec((3 * EMB, EMB), lambda i: (0, 0)),
        pl.BlockSpec((1, EMB), lambda i: (0, 0)),
        pl.BlockSpec((EMB, LAT), lambda i: (0, 0)),
        pl.BlockSpec((LAT, 1), lambda i: (0, 0)),
    ]
    args = [xg, xb, xp, W_proj, b_proj.reshape(1, EMB), W_fc, b_fc.reshape(LAT, 1)]
    body = _proj_body
    aliases = {}
    if acc is not None:
        specs.append(pl.BlockSpec(memory_space=pl.ANY))
        args.append(acc)
        body = _proj_body_acc
        aliases = {7: 0}
    blk_base = col_base // blk
    return pl.pallas_call(
        body,
        grid=(nblk,),
        in_specs=specs,
        out_specs=pl.BlockSpec((LAT, blk), lambda i, b=blk_base: (0, b + i)),
        out_shape=jax.ShapeDtypeStruct((LAT, B), jnp.float32),
        input_output_aliases=aliases,
    )(*args)


def kernel(gains, bws, pms, gain_table, bw_table, pm_table, W_proj, b_proj, W_fc, b_fc):
    g = gains.astype(jnp.int32)
    bw = bws.astype(jnp.int32)
    pm = pms.astype(jnp.int32)
    chunks = []
    off = 0
    for bc in CHUNK_ROWS:
        chunks.append((off, _make_gather3(off, bc)(g, bw, pm, gain_table, bw_table, pm_table)))
        off += bc
    acc = None
    for off, (xg, xb, xp) in chunks:
        acc = _proj_chunk(off, xg, xb, xp, W_proj, b_proj, W_fc, b_fc, acc)
    return acc.T
